# baseline (speedup 1.0000x reference)
import functools

import jax
import jax.numpy as jnp
from jax import lax
from jax.experimental import pallas as pl
from jax.experimental.pallas import tpu as pltpu
from jax.experimental.pallas import tpu_sc as plsc

NUM_FEATS = 11
NUM_CATS = 21
HIDDEN = 128
LANES = 16

_info = plsc.get_sparse_core_info()
NC, NS = _info.num_cores, _info.num_subcores
NW = NC * NS

TABLE_WORDS = NUM_FEATS * NUM_CATS * HIDDEN
CHUNK = 128
ROWS_PER_W = 3200
NPAD = ROWS_PER_W * NW
NCHUNKS = ROWS_PER_W // CHUNK


def _body(xf_hbm, embf_hbm, outf_hbm, tab_v, xc_v, outc_v):
    wid = lax.axis_index("s") * NC + lax.axis_index("c")
    base = wid * ROWS_PER_W
    pltpu.sync_copy(embf_hbm, tab_v)

    iota = lax.iota(jnp.int32, LANES)
    xstride_iota = iota * NUM_FEATS
    row_iota = iota * HIDDEN

    def chunk_body(ci, carry):
        row0 = base + ci * CHUNK
        pltpu.sync_copy(
            xf_hbm.at[pl.ds(row0 * NUM_FEATS, CHUNK * NUM_FEATS)], xc_v)

        for rg in range(CHUNK // LANES):
            bidx = []
            for i in range(NUM_FEATS):
                vcat = plsc.load_gather(
                    xc_v, [xstride_iota + (rg * LANES * NUM_FEATS + i)])
                bidx.append(vcat * HIDDEN + i * (NUM_CATS * HIDDEN))
            sbase = row_iota + rg * LANES * HIDDEN

            def h_body(h, carry2):
                acc = plsc.load_gather(tab_v, [bidx[0] + h])
                for i in range(1, NUM_FEATS):
                    acc = acc + plsc.load_gather(tab_v, [bidx[i] + h])
                plsc.store_scatter(outc_v, [sbase + h], acc)
                return carry2

            lax.fori_loop(0, HIDDEN, h_body, 0)

        pltpu.sync_copy(outc_v, outf_hbm.at[pl.ds(row0 * HIDDEN, CHUNK * HIDDEN)])
        return carry

    lax.fori_loop(0, NCHUNKS, chunk_body, 0)


@functools.partial(
    pl.kernel,
    mesh=plsc.VectorSubcoreMesh(core_axis_name="c", subcore_axis_name="s"),
    compiler_params=pltpu.CompilerParams(needs_layout_passes=False),
    out_type=jax.ShapeDtypeStruct((NPAD * HIDDEN,), jnp.float32),
    scratch_types=[
        pltpu.VMEM((TABLE_WORDS,), jnp.float32),
        pltpu.VMEM((CHUNK * NUM_FEATS,), jnp.int32),
        pltpu.VMEM((CHUNK * HIDDEN,), jnp.float32),
    ],
)
def _sc_encode(xf_hbm, embf_hbm, outf_hbm, tab_v, xc_v, outc_v):
    _body(xf_hbm, embf_hbm, outf_hbm, tab_v, xc_v, outc_v)


def kernel(x, emb):
    n = x.shape[0]
    xp = jnp.pad(x.astype(jnp.int32), ((0, NPAD - n), (0, 0)))
    outf = _sc_encode(xp.reshape(-1), emb.reshape(-1))
    return outf.reshape(NPAD, HIDDEN)[:n]


# speedup vs baseline: 1.4006x; 1.4006x over previous
import functools

import jax
import jax.numpy as jnp
from jax import lax
from jax.experimental import pallas as pl
from jax.experimental.pallas import tpu as pltpu
from jax.experimental.pallas import tpu_sc as plsc

NUM_FEATS = 11
NUM_CATS = 21
HIDDEN = 128
LANES = 16

_info = plsc.get_sparse_core_info()
NC, NS = _info.num_cores, _info.num_subcores
NW = NC * NS

TABLE_WORDS = NUM_FEATS * NUM_CATS * HIDDEN
CHUNK = 128
ROWS_PER_W = 3200
NPAD = ROWS_PER_W * NW
NCHUNKS = ROWS_PER_W // CHUNK


def _body(xf_hbm, embf_hbm, outf_hbm, tab_v, xc_v, outc_v):
    wid = lax.axis_index("s") * NC + lax.axis_index("c")
    base = wid * ROWS_PER_W
    pltpu.sync_copy(embf_hbm, tab_v)

    iota = lax.iota(jnp.int32, LANES)
    xstride_iota = iota * NUM_FEATS
    row_iota = iota * HIDDEN

    def chunk_body(ci, carry):
        row0 = base + ci * CHUNK
        pltpu.sync_copy(
            xf_hbm.at[pl.ds(row0 * NUM_FEATS, CHUNK * NUM_FEATS)], xc_v)

        for rg in range(CHUNK // LANES):
            bidx = []
            for i in range(NUM_FEATS):
                vcat = plsc.load_gather(
                    xc_v, [xstride_iota + (rg * LANES * NUM_FEATS + i)])
                bidx.append(vcat * HIDDEN + i * (NUM_CATS * HIDDEN))
            sbase = row_iota + rg * LANES * HIDDEN

            @plsc.parallel_loop(0, HIDDEN, unroll=8)
            def h_body(h):
                acc = plsc.load_gather(tab_v, [bidx[0] + h])
                for i in range(1, NUM_FEATS):
                    acc = acc + plsc.load_gather(tab_v, [bidx[i] + h])
                plsc.store_scatter(outc_v, [sbase + h], acc)

        pltpu.sync_copy(outc_v, outf_hbm.at[pl.ds(row0 * HIDDEN, CHUNK * HIDDEN)])
        return carry

    lax.fori_loop(0, NCHUNKS, chunk_body, 0)


@functools.partial(
    pl.kernel,
    mesh=plsc.VectorSubcoreMesh(core_axis_name="c", subcore_axis_name="s"),
    compiler_params=pltpu.CompilerParams(needs_layout_passes=False),
    out_type=jax.ShapeDtypeStruct((NPAD * HIDDEN,), jnp.float32),
    scratch_types=[
        pltpu.VMEM((TABLE_WORDS,), jnp.float32),
        pltpu.VMEM((CHUNK * NUM_FEATS,), jnp.int32),
        pltpu.VMEM((CHUNK * HIDDEN,), jnp.float32),
    ],
)
def _sc_encode(xf_hbm, embf_hbm, outf_hbm, tab_v, xc_v, outc_v):
    _body(xf_hbm, embf_hbm, outf_hbm, tab_v, xc_v, outc_v)


def kernel(x, emb):
    n = x.shape[0]
    xp = jnp.pad(x.astype(jnp.int32), ((0, NPAD - n), (0, 0)))
    outf = _sc_encode(xp.reshape(-1), emb.reshape(-1))
    return outf.reshape(NPAD, HIDDEN)[:n]


# speedup vs baseline: 8.5486x; 6.1035x over previous
import functools

import jax
import jax.numpy as jnp
from jax import lax
from jax.experimental import pallas as pl
from jax.experimental.pallas import tpu as pltpu
from jax.experimental.pallas import tpu_sc as plsc

NUM_FEATS = 11
NUM_CATS = 21
HIDDEN = 128
LANES = 16

_info = plsc.get_sparse_core_info()
NC, NS = _info.num_cores, _info.num_subcores
NW = NC * NS

TABLE_WORDS = NUM_FEATS * NUM_CATS * HIDDEN
CHUNK = 128
ROWS_PER_W = 3200
NPAD = ROWS_PER_W * NW
NCHUNKS = ROWS_PER_W // CHUNK


def _body(xf_hbm, embf_hbm, outf_hbm, tab_v, xc_v, outc_v):
    wid = lax.axis_index("s") * NC + lax.axis_index("c")
    base = wid * ROWS_PER_W
    pltpu.sync_copy(embf_hbm, tab_v)

    def chunk_body(ci, carry):
        row0 = base + ci * CHUNK
        pltpu.sync_copy(
            xf_hbm.at[pl.ds(row0 * NUM_FEATS, CHUNK * NUM_FEATS)],
            xc_v.at[pl.ds(0, CHUNK * NUM_FEATS)])

        @plsc.parallel_loop(0, CHUNK, unroll=2)
        def row_body(r):
            catv = xc_v[pl.ds(r * NUM_FEATS, LANES)]
            starts = [catv[i] * HIDDEN + i * (NUM_CATS * HIDDEN)
                      for i in range(NUM_FEATS)]
            for h in range(HIDDEN // LANES):
                acc = tab_v[pl.ds(starts[0] + h * LANES, LANES)]
                for i in range(1, NUM_FEATS):
                    acc = acc + tab_v[pl.ds(starts[i] + h * LANES, LANES)]
                outc_v[pl.ds(r * HIDDEN + h * LANES, LANES)] = acc

        pltpu.sync_copy(outc_v, outf_hbm.at[pl.ds(row0 * HIDDEN, CHUNK * HIDDEN)])
        return carry

    lax.fori_loop(0, NCHUNKS, chunk_body, 0)


@functools.partial(
    pl.kernel,
    mesh=plsc.VectorSubcoreMesh(core_axis_name="c", subcore_axis_name="s"),
    compiler_params=pltpu.CompilerParams(needs_layout_passes=False),
    out_type=jax.ShapeDtypeStruct((NPAD * HIDDEN,), jnp.float32),
    scratch_types=[
        pltpu.VMEM((TABLE_WORDS,), jnp.float32),
        pltpu.VMEM((CHUNK * NUM_FEATS + LANES,), jnp.int32),
        pltpu.VMEM((CHUNK * HIDDEN,), jnp.float32),
    ],
)
def _sc_encode(xf_hbm, embf_hbm, outf_hbm, tab_v, xc_v, outc_v):
    _body(xf_hbm, embf_hbm, outf_hbm, tab_v, xc_v, outc_v)


def kernel(x, emb):
    n = x.shape[0]
    xp = jnp.pad(x.astype(jnp.int32), ((0, NPAD - n), (0, 0)))
    outf = _sc_encode(xp.reshape(-1), emb.reshape(-1))
    return outf.reshape(NPAD, HIDDEN)[:n]


# speedup vs baseline: 8.7311x; 1.0214x over previous
import functools

import jax
import jax.numpy as jnp
from jax import lax
from jax.experimental import pallas as pl
from jax.experimental.pallas import tpu as pltpu
from jax.experimental.pallas import tpu_sc as plsc

NUM_FEATS = 11
NUM_CATS = 21
HIDDEN = 128
LANES = 16

_info = plsc.get_sparse_core_info()
NC, NS = _info.num_cores, _info.num_subcores
NW = NC * NS

TABLE_WORDS = NUM_FEATS * NUM_CATS * HIDDEN
CHUNK = 200
ROWS_PER_W = 3200
NPAD = ROWS_PER_W * NW
NCHUNKS = ROWS_PER_W // CHUNK
OUT_WORDS = CHUNK * HIDDEN


def _compute_chunk(tab_v, xc_v, outc_v):
    @plsc.parallel_loop(0, CHUNK, unroll=4)
    def row_body(r):
        catv = xc_v[pl.ds(r * NUM_FEATS, LANES)]
        starts = [catv[i] * HIDDEN + i * (NUM_CATS * HIDDEN)
                  for i in range(NUM_FEATS)]
        for h in range(HIDDEN // LANES):
            acc = tab_v[pl.ds(starts[0] + h * LANES, LANES)]
            for i in range(1, NUM_FEATS):
                acc = acc + tab_v[pl.ds(starts[i] + h * LANES, LANES)]
            outc_v[pl.ds(r * HIDDEN + h * LANES, LANES)] = acc


def _body(xf_hbm, embf_hbm, outf_hbm, tab_v, xc_v, outc0, outc1, sem0, sem1):
    wid = lax.axis_index("s") * NC + lax.axis_index("c")
    base = wid * ROWS_PER_W
    outcs = (outc0, outc1)
    sems = (sem0, sem1)

    pltpu.sync_copy(embf_hbm, tab_v)

    for b in range(2):
        pltpu.make_async_copy(
            embf_hbm.at[pl.ds(0, OUT_WORDS)], outcs[b], sems[b]).start()

    def pair_body(ii, carry):
        for b in range(2):
            c = ii * 2 + b
            row0 = base + c * CHUNK
            out_slice = outf_hbm.at[pl.ds(row0 * HIDDEN, OUT_WORDS)]
            pltpu.sync_copy(
                xf_hbm.at[pl.ds(row0 * NUM_FEATS, CHUNK * NUM_FEATS)],
                xc_v.at[pl.ds(0, CHUNK * NUM_FEATS)])
            pltpu.make_async_copy(outcs[b], out_slice, sems[b]).wait()
            _compute_chunk(tab_v, xc_v, outcs[b])
            pltpu.make_async_copy(outcs[b], out_slice, sems[b]).start()
        return carry

    lax.fori_loop(0, NCHUNKS // 2, pair_body, 0)

    for b in range(2):
        c = NCHUNKS - 2 + b
        row0 = base + c * CHUNK
        pltpu.make_async_copy(
            outcs[b], outf_hbm.at[pl.ds(row0 * HIDDEN, OUT_WORDS)],
            sems[b]).wait()


@functools.partial(
    pl.kernel,
    mesh=plsc.VectorSubcoreMesh(core_axis_name="c", subcore_axis_name="s"),
    compiler_params=pltpu.CompilerParams(needs_layout_passes=False),
    out_type=jax.ShapeDtypeStruct((NPAD * HIDDEN,), jnp.float32),
    scratch_types=[
        pltpu.VMEM((TABLE_WORDS,), jnp.float32),
        pltpu.VMEM((CHUNK * NUM_FEATS + LANES,), jnp.int32),
        pltpu.VMEM((OUT_WORDS,), jnp.float32),
        pltpu.VMEM((OUT_WORDS,), jnp.float32),
        pltpu.SemaphoreType.DMA,
        pltpu.SemaphoreType.DMA,
    ],
)
def _sc_encode(xf_hbm, embf_hbm, outf_hbm, tab_v, xc_v, outc0, outc1,
               sem0, sem1):
    _body(xf_hbm, embf_hbm, outf_hbm, tab_v, xc_v, outc0, outc1, sem0, sem1)


def kernel(x, emb):
    n = x.shape[0]
    xp = jnp.pad(x.astype(jnp.int32), ((0, NPAD - n), (0, 0)))
    outf = _sc_encode(xp.reshape(-1), emb.reshape(-1))
    return outf.reshape(NPAD, HIDDEN)[:n]
